# Initial kernel scaffold; baseline (speedup 1.0000x reference)
import jax
import jax.numpy as jnp
from jax.experimental import pallas as pl


def kernel(inputs, table):
    raise NotImplementedError("write your pallas kernel here")


# speedup vs baseline: 1.7484x; 1.7484x over previous
import functools

import jax
import jax.numpy as jnp
from jax import lax
from jax.experimental import pallas as pl
from jax.experimental.pallas import tpu as pltpu
from jax.experimental.pallas import tpu_sc as plsc

VOCAB = 100000
D = 128
B = 16384
T = 6
BF = B * T
NC, NS, L = 2, 16, 16
NW = NC * NS
BPW = BF // NW
CH = 256
NCH = BPW // CH
NVD = D // L


def _sc_body(idx_hbm, table_hbm, out_hbm, idx_v, rows_v, sem):
    wid = lax.axis_index("s") * NC + lax.axis_index("c")
    base = wid * BPW
    pltpu.sync_copy(idx_hbm.at[pl.ds(base, BPW)], idx_v)

    def chunk(c, carry):
        row0 = pl.multiple_of(c * CH, CH)
        pltpu.async_copy(
            table_hbm.at[idx_v.at[pl.ds(row0, CH)]], rows_v, sem
        ).wait()

        def relu_row(r, carry2):
            for j in range(NVD):
                v = rows_v[r, pl.ds(j * L, L)]
                rows_v[r, pl.ds(j * L, L)] = jnp.maximum(v, 0.0)
            return carry2

        lax.fori_loop(0, CH, relu_row, 0)
        pltpu.sync_copy(rows_v, out_hbm.at[pl.ds(base + row0, CH)])
        return carry

    lax.fori_loop(0, NCH, chunk, 0)


@jax.jit
def _sc_gather_relu(idx, table):
    mesh = plsc.VectorSubcoreMesh(core_axis_name="c", subcore_axis_name="s")
    f = functools.partial(
        pl.kernel,
        mesh=mesh,
        out_type=jax.ShapeDtypeStruct((BF, D), jnp.float32),
        scratch_types=[
            pltpu.VMEM((BPW,), jnp.int32),
            pltpu.VMEM((CH, D), jnp.float32),
            pltpu.SemaphoreType.DMA,
        ],
    )(_sc_body)
    return f(idx, table)


def kernel(inputs, table):
    idx = inputs.reshape(BF)
    out = _sc_gather_relu(idx, table)
    return out.reshape(B, T * D)


# speedup vs baseline: 2.0274x; 1.1595x over previous
import functools

import jax
import jax.numpy as jnp
from jax import lax
from jax.experimental import pallas as pl
from jax.experimental.pallas import tpu as pltpu
from jax.experimental.pallas import tpu_sc as plsc

VOCAB = 100000
D = 128
B = 16384
T = 6
BF = B * T
NC, NS, L = 2, 16, 16
NW = NC * NS
BPW = BF // NW
CH = 256
NCH = BPW // CH
NVD = D // L


def _sc_body(idx_hbm, table_hbm, out_hbm, idx_v, rows_v, g0, g1, o0, o1):
    wid = lax.axis_index("s") * NC + lax.axis_index("c")
    base = wid * BPW
    pltpu.sync_copy(idx_hbm.at[pl.ds(base, BPW)], idx_v)
    gsem = (g0, g1)
    osem = (o0, o1)

    def start_gather(c, b):
        return pltpu.async_copy(
            table_hbm.at[idx_v.at[pl.ds(c * CH, CH)]], rows_v.at[b], gsem[b]
        )

    def relu_buf(b):
        def relu_row(r, carry):
            for j in range(NVD):
                v = rows_v[b, r, pl.ds(j * L, L)]
                rows_v[b, r, pl.ds(j * L, L)] = jnp.maximum(v, 0.0)
            return carry

        lax.fori_loop(0, CH, relu_row, 0)

    out_h = [None, None]
    g_h = [None, None]
    g_h[0] = start_gather(0, 0)
    for c in range(NCH):
        b = c & 1
        if c + 1 < NCH:
            nb = (c + 1) & 1
            if out_h[nb] is not None:
                out_h[nb].wait()
                out_h[nb] = None
            g_h[nb] = start_gather(c + 1, nb)
        g_h[b].wait()
        relu_buf(b)
        out_h[b] = pltpu.async_copy(
            rows_v.at[b], out_hbm.at[pl.ds(base + c * CH, CH)], osem[b]
        )
    for h in out_h:
        if h is not None:
            h.wait()


@jax.jit
def _sc_gather_relu(idx, table):
    mesh = plsc.VectorSubcoreMesh(core_axis_name="c", subcore_axis_name="s")
    f = functools.partial(
        pl.kernel,
        mesh=mesh,
        out_type=jax.ShapeDtypeStruct((BF, D), jnp.float32),
        scratch_types=[
            pltpu.VMEM((BPW,), jnp.int32),
            pltpu.VMEM((2, CH, D), jnp.float32),
            pltpu.SemaphoreType.DMA,
            pltpu.SemaphoreType.DMA,
            pltpu.SemaphoreType.DMA,
            pltpu.SemaphoreType.DMA,
        ],
    )(_sc_body)
    return f(idx, table)


def kernel(inputs, table):
    idx = inputs.reshape(BF)
    out = _sc_gather_relu(idx, table)
    return out.reshape(B, T * D)
